# Initial kernel scaffold; baseline (speedup 1.0000x reference)
import jax
import jax.numpy as jnp
from jax.experimental import pallas as pl


def kernel(x, y, t2v_w, t2v_b, y_emb_W, y_emb_b, var_table, given_table):
    raise NotImplementedError("write your pallas kernel here")


# speedup vs baseline: 3.9758x; 3.9758x over previous
import functools

import jax
import jax.numpy as jnp
from jax import lax
from jax.experimental import pallas as pl
from jax.experimental.pallas import tpu as pltpu


def _emb_body(x_ref, yt_ref, w_ref, b_ref, W_ref, bias_ref, vt_ref, gt_ref,
              vte_ref, vemb_ref, vidx_ref, t_scr, *, L, DX, TD, DM):
    d = pl.program_id(1)

    @pl.when(d == 0)
    def _compute_t():
        acc = jnp.zeros((L, DM), jnp.float32)
        lane = lax.broadcasted_iota(jnp.int32, (1, TD), 1)
        for j in range(DX + 1):
            if j < DX:
                col = x_ref[0, :, j:j + 1]
                col = jnp.where(jnp.isnan(col), 0.0, col)
            else:
                col = (lax.broadcasted_iota(jnp.int32, (L, 1), 0)
                       .astype(jnp.float32) * (1.0 / L))
            aff = col * w_ref[j:j + 1, :] + b_ref[j:j + 1, :]
            tj = jnp.where(lane == 0, aff, jnp.sin(aff))
            acc = acc + jnp.dot(tj, W_ref[1 + j * TD:1 + (j + 1) * TD, :],
                                preferred_element_type=jnp.float32)
        t_scr[...] = acc + bias_ref[...] + gt_ref[1:2, :]

    yc = yt_ref[0, 0]
    m = jnp.isnan(yc)
    yclean = jnp.where(m, 0.0, yc)
    w0 = W_ref[0:1, :]
    delta = gt_ref[0:1, :] - gt_ref[1:2, :]
    vte_ref[0, 0] = (yclean * w0 + t_scr[...]
                     + m.astype(jnp.float32) * delta)
    vemb_ref[0, 0] = jnp.broadcast_to(vt_ref[pl.ds(d, 1), :], (L, DM))
    vidx_ref[...] = jnp.full((1, 1, 1, L), d, dtype=jnp.int32)


def kernel(x, y, t2v_w, t2v_b, y_emb_W, y_emb_b, var_table, given_table):
    B, L, DX = x.shape
    DY = y.shape[2]
    DM = y_emb_W.shape[1]
    TD = t2v_w.shape[1]

    yt = jnp.transpose(y, (0, 2, 1)).reshape(B, DY, L, 1)
    bias2 = y_emb_b.reshape(1, DM)

    body = functools.partial(_emb_body, L=L, DX=DX, TD=TD, DM=DM)
    vte, vemb, vidx = pl.pallas_call(
        body,
        grid=(B, DY),
        in_specs=[
            pl.BlockSpec((1, L, DX), lambda b, d: (b, 0, 0)),
            pl.BlockSpec((1, 1, L, 1), lambda b, d: (b, d, 0, 0)),
            pl.BlockSpec((DX + 1, TD), lambda b, d: (0, 0)),
            pl.BlockSpec((DX + 1, TD), lambda b, d: (0, 0)),
            pl.BlockSpec((1 + (DX + 1) * TD, DM), lambda b, d: (0, 0)),
            pl.BlockSpec((1, DM), lambda b, d: (0, 0)),
            pl.BlockSpec((DY, DM), lambda b, d: (0, 0)),
            pl.BlockSpec((2, DM), lambda b, d: (0, 0)),
        ],
        out_specs=[
            pl.BlockSpec((1, 1, L, DM), lambda b, d: (b, d, 0, 0)),
            pl.BlockSpec((1, 1, L, DM), lambda b, d: (b, d, 0, 0)),
            pl.BlockSpec((1, 1, 1, L), lambda b, d: (b, d, 0, 0)),
        ],
        out_shape=[
            jax.ShapeDtypeStruct((B, DY, L, DM), jnp.float32),
            jax.ShapeDtypeStruct((B, DY, L, DM), jnp.float32),
            jax.ShapeDtypeStruct((B, DY, 1, L), jnp.int32),
        ],
        scratch_shapes=[pltpu.VMEM((L, DM), jnp.float32)],
        compiler_params=pltpu.CompilerParams(
            dimension_semantics=("arbitrary", "arbitrary")),
    )(x, yt, t2v_w, t2v_b, y_emb_W, bias2, var_table, given_table)

    return (vte.reshape(B, DY * L, DM),
            vemb.reshape(B, DY * L, DM),
            vidx.reshape(B, DY * L))


# speedup vs baseline: 6.0053x; 1.5105x over previous
import functools

import jax
import jax.numpy as jnp
from jax import lax
from jax.experimental import pallas as pl
from jax.experimental.pallas import tpu as pltpu

_G = 4


def _emb_body(x_ref, yt_ref, w_ref, b_ref, W_ref, bias_ref, vt_ref, gt_ref,
              vte_ref, vemb_ref, vidx_ref, t_scr, *, L, DX, TD, DM):
    g = pl.program_id(1)

    @pl.when(g == 0)
    def _compute_t():
        acc = jnp.zeros((L, DM), jnp.float32)
        lane = lax.broadcasted_iota(jnp.int32, (1, TD), 1)
        for j in range(DX + 1):
            if j < DX:
                col = x_ref[0, :, j:j + 1]
                col = jnp.where(jnp.isnan(col), 0.0, col)
            else:
                col = (lax.broadcasted_iota(jnp.int32, (L, 1), 0)
                       .astype(jnp.float32) * (1.0 / L))
            aff = col * w_ref[j:j + 1, :] + b_ref[j:j + 1, :]
            tj = jnp.where(lane == 0, aff, jnp.sin(aff))
            acc = acc + jnp.dot(tj, W_ref[1 + j * TD:1 + (j + 1) * TD, :],
                                preferred_element_type=jnp.float32)
        t_scr[...] = acc + bias_ref[...] + gt_ref[1:2, :]

    w0 = W_ref[0:1, :]
    delta = gt_ref[0:1, :] - gt_ref[1:2, :]
    t_val = t_scr[...]
    for i in range(_G):
        yc = yt_ref[0, i]
        m = jnp.isnan(yc)
        yclean = jnp.where(m, 0.0, yc)
        vte_ref[0, i] = (yclean * w0 + t_val
                         + m.astype(jnp.float32) * delta)
        d = g * _G + i
        vemb_ref[0, i] = jnp.broadcast_to(vt_ref[pl.ds(d, 1), :], (L, DM))
        vidx_ref[0, i] = jnp.full((1, L), d, dtype=jnp.int32)


def kernel(x, y, t2v_w, t2v_b, y_emb_W, y_emb_b, var_table, given_table):
    B, L, DX = x.shape
    DY = y.shape[2]
    DM = y_emb_W.shape[1]
    TD = t2v_w.shape[1]

    yt = jnp.transpose(y, (0, 2, 1)).reshape(B, DY, L, 1)
    bias2 = y_emb_b.reshape(1, DM)

    body = functools.partial(_emb_body, L=L, DX=DX, TD=TD, DM=DM)
    vte, vemb, vidx = pl.pallas_call(
        body,
        grid=(B, DY // _G),
        in_specs=[
            pl.BlockSpec((1, L, DX), lambda b, g: (b, 0, 0)),
            pl.BlockSpec((1, _G, L, 1), lambda b, g: (b, g, 0, 0)),
            pl.BlockSpec((DX + 1, TD), lambda b, g: (0, 0)),
            pl.BlockSpec((DX + 1, TD), lambda b, g: (0, 0)),
            pl.BlockSpec((1 + (DX + 1) * TD, DM), lambda b, g: (0, 0)),
            pl.BlockSpec((1, DM), lambda b, g: (0, 0)),
            pl.BlockSpec((DY, DM), lambda b, g: (0, 0)),
            pl.BlockSpec((2, DM), lambda b, g: (0, 0)),
        ],
        out_specs=[
            pl.BlockSpec((1, _G, L, DM), lambda b, g: (b, g, 0, 0)),
            pl.BlockSpec((1, _G, L, DM), lambda b, g: (b, g, 0, 0)),
            pl.BlockSpec((1, _G, 1, L), lambda b, g: (b, g, 0, 0)),
        ],
        out_shape=[
            jax.ShapeDtypeStruct((B, DY, L, DM), jnp.float32),
            jax.ShapeDtypeStruct((B, DY, L, DM), jnp.float32),
            jax.ShapeDtypeStruct((B, DY, 1, L), jnp.int32),
        ],
        scratch_shapes=[pltpu.VMEM((L, DM), jnp.float32)],
        compiler_params=pltpu.CompilerParams(
            dimension_semantics=("arbitrary", "arbitrary")),
    )(x, yt, t2v_w, t2v_b, y_emb_W, bias2, var_table, given_table)

    return (vte.reshape(B, DY * L, DM),
            vemb.reshape(B, DY * L, DM),
            vidx.reshape(B, DY * L))


# speedup vs baseline: 6.6029x; 1.0995x over previous
import functools

import jax
import jax.numpy as jnp
from jax import lax
from jax.experimental import pallas as pl
from jax.experimental.pallas import tpu as pltpu

_G = 8


def _emb_body(x_ref, yt_ref, w_ref, b_ref, W_ref, bias_ref, vt_ref, gt_ref,
              vte_ref, vemb_ref, vidx_ref, t_scr, *, L, DX, TD, DM):
    g = pl.program_id(1)

    @pl.when(g == 0)
    def _compute_t():
        acc = jnp.zeros((L, DM), jnp.float32)
        lane = lax.broadcasted_iota(jnp.int32, (1, TD), 1)
        for j in range(DX + 1):
            if j < DX:
                col = x_ref[0, :, j:j + 1]
                col = jnp.where(jnp.isnan(col), 0.0, col)
            else:
                col = (lax.broadcasted_iota(jnp.int32, (L, 1), 0)
                       .astype(jnp.float32) * (1.0 / L))
            aff = col * w_ref[j:j + 1, :] + b_ref[j:j + 1, :]
            tj = jnp.where(lane == 0, aff, jnp.sin(aff))
            acc = acc + jnp.dot(tj, W_ref[1 + j * TD:1 + (j + 1) * TD, :],
                                preferred_element_type=jnp.float32)
        t_scr[...] = acc + bias_ref[...] + gt_ref[1:2, :]

    w0 = W_ref[0:1, :]
    delta = gt_ref[0:1, :] - gt_ref[1:2, :]
    t_val = t_scr[...]
    for i in range(_G):
        yc = yt_ref[0, i]
        m = jnp.isnan(yc)
        yclean = jnp.where(m, 0.0, yc)
        vte_ref[0, i] = (yclean * w0 + t_val
                         + m.astype(jnp.float32) * delta)
        d = g * _G + i
        vemb_ref[0, i] = jnp.broadcast_to(vt_ref[pl.ds(d, 1), :], (L, DM))
        vidx_ref[0, i] = jnp.full((1, L), d, dtype=jnp.int32)


def kernel(x, y, t2v_w, t2v_b, y_emb_W, y_emb_b, var_table, given_table):
    B, L, DX = x.shape
    DY = y.shape[2]
    DM = y_emb_W.shape[1]
    TD = t2v_w.shape[1]

    yt = jnp.transpose(y, (0, 2, 1)).reshape(B, DY, L, 1)
    bias2 = y_emb_b.reshape(1, DM)

    body = functools.partial(_emb_body, L=L, DX=DX, TD=TD, DM=DM)
    vte, vemb, vidx = pl.pallas_call(
        body,
        grid=(B, DY // _G),
        in_specs=[
            pl.BlockSpec((1, L, DX), lambda b, g: (b, 0, 0)),
            pl.BlockSpec((1, _G, L, 1), lambda b, g: (b, g, 0, 0)),
            pl.BlockSpec((DX + 1, TD), lambda b, g: (0, 0)),
            pl.BlockSpec((DX + 1, TD), lambda b, g: (0, 0)),
            pl.BlockSpec((1 + (DX + 1) * TD, DM), lambda b, g: (0, 0)),
            pl.BlockSpec((1, DM), lambda b, g: (0, 0)),
            pl.BlockSpec((DY, DM), lambda b, g: (0, 0)),
            pl.BlockSpec((2, DM), lambda b, g: (0, 0)),
        ],
        out_specs=[
            pl.BlockSpec((1, _G, L, DM), lambda b, g: (b, g, 0, 0)),
            pl.BlockSpec((1, _G, L, DM), lambda b, g: (b, g, 0, 0)),
            pl.BlockSpec((1, _G, 1, L), lambda b, g: (b, g, 0, 0)),
        ],
        out_shape=[
            jax.ShapeDtypeStruct((B, DY, L, DM), jnp.float32),
            jax.ShapeDtypeStruct((B, DY, L, DM), jnp.float32),
            jax.ShapeDtypeStruct((B, DY, 1, L), jnp.int32),
        ],
        scratch_shapes=[pltpu.VMEM((L, DM), jnp.float32)],
        compiler_params=pltpu.CompilerParams(
            dimension_semantics=("arbitrary", "arbitrary")),
    )(x, yt, t2v_w, t2v_b, y_emb_W, bias2, var_table, given_table)

    return (vte.reshape(B, DY * L, DM),
            vemb.reshape(B, DY * L, DM),
            vidx.reshape(B, DY * L))


# speedup vs baseline: 7.4225x; 1.1241x over previous
import functools

import jax
import jax.numpy as jnp
from jax import lax
from jax.experimental import pallas as pl
from jax.experimental.pallas import tpu as pltpu

_G = 16


def _emb_body(x_ref, yt_ref, w_ref, b_ref, W_ref, bias_ref, vt_ref, gt_ref,
              vte_ref, vemb_ref, vidx_ref, t_scr, *, L, DX, TD, DM):
    g = pl.program_id(1)

    @pl.when(g == 0)
    def _compute_t():
        acc = jnp.zeros((L, DM), jnp.float32)
        lane = lax.broadcasted_iota(jnp.int32, (1, TD), 1)
        for j in range(DX + 1):
            if j < DX:
                col = x_ref[0, :, j:j + 1]
                col = jnp.where(jnp.isnan(col), 0.0, col)
            else:
                col = (lax.broadcasted_iota(jnp.int32, (L, 1), 0)
                       .astype(jnp.float32) * (1.0 / L))
            aff = col * w_ref[j:j + 1, :] + b_ref[j:j + 1, :]
            tj = jnp.where(lane == 0, aff, jnp.sin(aff))
            acc = acc + jnp.dot(tj, W_ref[1 + j * TD:1 + (j + 1) * TD, :],
                                preferred_element_type=jnp.float32)
        t_scr[...] = acc + bias_ref[...] + gt_ref[1:2, :]

    w0 = W_ref[0:1, :]
    delta = gt_ref[0:1, :] - gt_ref[1:2, :]
    t_val = t_scr[...]
    for i in range(_G):
        yc = yt_ref[0, i]
        m = jnp.isnan(yc)
        yclean = jnp.where(m, 0.0, yc)
        vte_ref[0, i] = (yclean * w0 + t_val
                         + m.astype(jnp.float32) * delta)
        d = g * _G + i
        vemb_ref[0, i] = jnp.broadcast_to(vt_ref[pl.ds(d, 1), :], (L, DM))
        vidx_ref[0, i] = jnp.full((1, L), d, dtype=jnp.int32)


def kernel(x, y, t2v_w, t2v_b, y_emb_W, y_emb_b, var_table, given_table):
    B, L, DX = x.shape
    DY = y.shape[2]
    DM = y_emb_W.shape[1]
    TD = t2v_w.shape[1]

    yt = jnp.transpose(y, (0, 2, 1)).reshape(B, DY, L, 1)
    bias2 = y_emb_b.reshape(1, DM)

    body = functools.partial(_emb_body, L=L, DX=DX, TD=TD, DM=DM)
    vte, vemb, vidx = pl.pallas_call(
        body,
        grid=(B, DY // _G),
        in_specs=[
            pl.BlockSpec((1, L, DX), lambda b, g: (b, 0, 0)),
            pl.BlockSpec((1, _G, L, 1), lambda b, g: (b, g, 0, 0)),
            pl.BlockSpec((DX + 1, TD), lambda b, g: (0, 0)),
            pl.BlockSpec((DX + 1, TD), lambda b, g: (0, 0)),
            pl.BlockSpec((1 + (DX + 1) * TD, DM), lambda b, g: (0, 0)),
            pl.BlockSpec((1, DM), lambda b, g: (0, 0)),
            pl.BlockSpec((DY, DM), lambda b, g: (0, 0)),
            pl.BlockSpec((2, DM), lambda b, g: (0, 0)),
        ],
        out_specs=[
            pl.BlockSpec((1, _G, L, DM), lambda b, g: (b, g, 0, 0)),
            pl.BlockSpec((1, _G, L, DM), lambda b, g: (b, g, 0, 0)),
            pl.BlockSpec((1, _G, 1, L), lambda b, g: (b, g, 0, 0)),
        ],
        out_shape=[
            jax.ShapeDtypeStruct((B, DY, L, DM), jnp.float32),
            jax.ShapeDtypeStruct((B, DY, L, DM), jnp.float32),
            jax.ShapeDtypeStruct((B, DY, 1, L), jnp.int32),
        ],
        scratch_shapes=[pltpu.VMEM((L, DM), jnp.float32)],
        compiler_params=pltpu.CompilerParams(
            dimension_semantics=("arbitrary", "arbitrary")),
    )(x, yt, t2v_w, t2v_b, y_emb_W, bias2, var_table, given_table)

    return (vte.reshape(B, DY * L, DM),
            vemb.reshape(B, DY * L, DM),
            vidx.reshape(B, DY * L))
